# baseline (speedup 1.0000x reference)
import jax
import jax.numpy as jnp
from jax import lax
from jax.experimental import pallas as pl
from jax.experimental.pallas import tpu as pltpu
from jax.experimental.pallas import tpu_sc as plsc

_B, _S, _D = 4, 8192, 1024
_NC, _NS = 2, 16
_NW = _NC * _NS
_SPW = _S // _NW
_CH = 16
_NCHUNK = _SPW // _CH
_CHW = _CH * _D


def _sc_body(x_hbm, emb_hbm, out_hbm, ebuf, xbuf):
    wid = lax.axis_index("s") * _NC + lax.axis_index("c")
    s0 = wid * _SPW

    def chunk_body(c, carry):
        eoff = (s0 + c * _CH) * _D
        pltpu.sync_copy(emb_hbm.at[pl.ds(eoff, _CHW)], ebuf)
        for b in range(_B):
            xoff = (b * _S + s0 + c * _CH) * _D
            pltpu.sync_copy(x_hbm.at[pl.ds(xoff, _CHW)], xbuf)

            @plsc.parallel_loop(0, _CHW, 16, unroll=8)
            def add_body(i):
                xbuf[pl.ds(i, 16)] = xbuf[pl.ds(i, 16)] + ebuf[pl.ds(i, 16)]

            pltpu.sync_copy(xbuf, out_hbm.at[pl.ds(xoff, _CHW)])
        return carry

    lax.fori_loop(0, _NCHUNK, chunk_body, 0)


def kernel(x, emb_table):
    B, S, D = x.shape
    mesh = plsc.VectorSubcoreMesh(core_axis_name="c", subcore_axis_name="s")
    out_flat = pl.kernel(
        _sc_body,
        out_type=jax.ShapeDtypeStruct((B * S * D,), jnp.float32),
        mesh=mesh,
        scratch_types=[
            pltpu.VMEM((_CHW,), jnp.float32),
            pltpu.VMEM((_CHW,), jnp.float32),
        ],
    )(x.reshape(-1), emb_table.reshape(-1))
    return out_flat.reshape(B, S, D)


# speedup vs baseline: 1.3254x; 1.3254x over previous
import jax
import jax.numpy as jnp
from jax import lax
from jax.experimental import pallas as pl
from jax.experimental.pallas import tpu as pltpu
from jax.experimental.pallas import tpu_sc as plsc

_B, _S, _D = 4, 8192, 1024
_NC, _NS = 2, 16
_NW = _NC * _NS
_SPW = _S // _NW
_CH = 8
_NCHUNK = _SPW // _CH
_CHW = _CH * _D


def _sc_body(x_hbm, emb_hbm, out_hbm,
             xb0, xb1, xb2, xb3, xb4, xb5, xb6, xb7, eb0, eb1,
             xsem, osem, esem):
    xb = (xb0, xb1, xb2, xb3, xb4, xb5, xb6, xb7)
    eb = (eb0, eb1)
    wid = lax.axis_index("s") * _NC + lax.axis_index("c")
    s0 = wid * _SPW

    def eoff(c):
        return (s0 + c * _CH) * _D

    def xoff(c, k):
        return (k * _S + s0 + c * _CH) * _D

    def ein_desc(c, par):
        return pltpu.make_async_copy(
            emb_hbm.at[pl.ds(eoff(c), _CHW)], eb[par], esem.at[par])

    def xin_desc(c, k, par):
        r = par * 4 + k
        return pltpu.make_async_copy(
            x_hbm.at[pl.ds(xoff(c, k), _CHW)], xb[r], xsem.at[r])

    def out_desc(c, k, par):
        r = par * 4 + k
        return pltpu.make_async_copy(
            xb[r], out_hbm.at[pl.ds(xoff(c, k), _CHW)], osem.at[r])

    def item(c, k, par, issue_ein_next=True, wait_out_prev=True,
             issue_xin_next=True):
        if k == 0 and issue_ein_next:
            ein_desc(c + 1, 1 - par).start()
        xin_desc(c, k, par).wait()
        if k == 0:
            ein_desc(c, par).wait()
        buf = xb[par * 4 + k]
        e = eb[par]

        @plsc.parallel_loop(0, _CHW, 16, unroll=8)
        def add_body(i):
            buf[pl.ds(i, 16)] = buf[pl.ds(i, 16)] + e[pl.ds(i, 16)]

        out_desc(c, k, par).start()
        if wait_out_prev:
            out_desc(c - 1, k, 1 - par).wait()
        if issue_xin_next:
            xin_desc(c + 1, k, 1 - par).start()

    ein_desc(0, 0).start()
    for k in range(_B):
        xin_desc(0, k, 0).start()
    for k in range(_B):
        item(0, k, 0, wait_out_prev=False)

    def pair_body(p, carry):
        c1 = 1 + 2 * p
        for k in range(_B):
            item(c1, k, 1)
        for k in range(_B):
            item(c1 + 1, k, 0)
        return carry

    lax.fori_loop(0, (_NCHUNK - 2) // 2, pair_body, 0)

    for k in range(_B):
        item(_NCHUNK - 1, k, 1, issue_ein_next=False, issue_xin_next=False)
    for k in range(_B):
        out_desc(_NCHUNK - 1, k, 1).wait()


def kernel(x, emb_table):
    B, S, D = x.shape
    mesh = plsc.VectorSubcoreMesh(core_axis_name="c", subcore_axis_name="s")
    out_flat = pl.kernel(
        _sc_body,
        out_type=jax.ShapeDtypeStruct((B * S * D,), jnp.float32),
        mesh=mesh,
        scratch_types=(
            [pltpu.VMEM((_CHW,), jnp.float32) for _ in range(8)]
            + [pltpu.VMEM((_CHW,), jnp.float32) for _ in range(2)]
            + [pltpu.SemaphoreType.DMA((8,)),
               pltpu.SemaphoreType.DMA((8,)),
               pltpu.SemaphoreType.DMA((2,))]
        ),
    )(x.reshape(-1), emb_table.reshape(-1))
    return out_flat.reshape(B, S, D)
